# Initial kernel scaffold; baseline (speedup 1.0000x reference)
import jax
import jax.numpy as jnp
from jax.experimental import pallas as pl


def kernel(ref, node, batch_index, W_lin, b_lin, W_alpha, b_alpha, gru_kernel, gru_rec, gru_bias):
    raise NotImplementedError("write your pallas kernel here")


# speedup vs baseline: 4.0129x; 4.0129x over previous
import jax
import jax.numpy as jnp
from jax.experimental import pallas as pl

UNITS = 512
F = 512
BATCH = 256
DEPTH = 3
BLK = 512
AW = 640


def _prep_body(node_ref, W_ref, b_ref, bi_ref, wn_ref, h0_ref):
    i = pl.program_id(0)
    x = node_ref[...]
    wn_ref[...] = jnp.dot(x, W_ref[...], preferred_element_type=jnp.float32, precision=jax.lax.Precision.HIGHEST) + b_ref[...]
    bi = bi_ref[0]
    oh = (jax.lax.broadcasted_iota(jnp.int32, (BATCH, BLK), 0) == bi).astype(jnp.float32)
    part = jnp.dot(oh, x, preferred_element_type=jnp.float32, precision=jax.lax.Precision.HIGHEST)

    @pl.when(i == 0)
    def _():
        h0_ref[...] = part

    @pl.when(i > 0)
    def _():
        h0_ref[...] += part


def _sh_body(h_ref, wa1_ref, out_ref):
    out_ref[...] = jnp.dot(h_ref[...], wa1_ref[...], preferred_element_type=jnp.float32, precision=jax.lax.Precision.HIGHEST)


def _iter_body(wn_ref, sn_ref, bi_ref, sh_ref, acc_ref):
    i = pl.program_id(0)
    bi = bi_ref[0]
    oh = (jax.lax.broadcasted_iota(jnp.int32, (BATCH, BLK), 0) == bi).astype(jnp.float32)
    shg = jnp.dot(sh_ref[...], oh, preferred_element_type=jnp.float32, precision=jax.lax.Precision.HIGHEST)
    av = shg[0:1] + sn_ref[0]
    av = jnp.where(av > 0, av, 0.2 * av)
    e = jnp.exp(av)
    ow = oh * e
    part = jnp.dot(ow, wn_ref[...], preferred_element_type=jnp.float32, precision=jax.lax.Precision.HIGHEST)

    @pl.when(i == 0)
    def _():
        acc_ref[...] = part

    @pl.when(i > 0)
    def _():
        acc_ref[...] += part


def _gru_body(acc_ref, h_ref, gk_ref, gr_ref, gb_ref, wa1_ref, hn_ref, shc_ref):
    acc = acc_ref[...]
    denom = jnp.maximum(acc[:, UNITS:UNITS + 1], 1e-30)
    cont = acc[:, :UNITS] / denom
    cont = jnp.where(cont > 0, cont, jnp.exp(cont) - 1.0)
    h = h_ref[...]
    mx = jnp.dot(cont, gk_ref[...], preferred_element_type=jnp.float32, precision=jax.lax.Precision.HIGHEST) + gb_ref[0:1]
    mi = jnp.dot(h, gr_ref[...], preferred_element_type=jnp.float32, precision=jax.lax.Precision.HIGHEST) + gb_ref[1:2]
    xz, xr, xh = mx[:, :UNITS], mx[:, UNITS:2 * UNITS], mx[:, 2 * UNITS:]
    rz, rr, rh = mi[:, :UNITS], mi[:, UNITS:2 * UNITS], mi[:, 2 * UNITS:]
    z = jax.nn.sigmoid(xz + rz)
    r = jax.nn.sigmoid(xr + rr)
    hh = jnp.tanh(xh + r * rh)
    hn = z * h + (1.0 - z) * hh
    hn_ref[...] = hn
    shc_ref[...] = jnp.dot(hn, wa1_ref[...], preferred_element_type=jnp.float32, precision=jax.lax.Precision.HIGHEST)


def kernel(ref, node, batch_index, W_lin, b_lin, W_alpha, b_alpha,
           gru_kernel, gru_rec, gru_bias):
    N = node.shape[0]
    NB = -(-N // BLK)
    NPAD = NB * BLK
    nodep = jnp.pad(node, ((0, NPAD - N), (0, 0)))
    bip = jnp.pad(batch_index.astype(jnp.int32), (0, NPAD - N),
                  constant_values=BATCH)
    bi3 = bip.reshape(NB, 1, BLK)

    W_aug = (jnp.zeros((F, AW), jnp.float32)
             .at[:, :UNITS].set(W_lin)
             .at[:, UNITS + 1].set(W_alpha[UNITS:, 0]))
    b_aug = (jnp.zeros((AW,), jnp.float32)
             .at[:UNITS].set(b_lin)
             .at[UNITS].set(1.0)
             .at[UNITS + 1].set(b_alpha[0])).reshape(1, AW)
    wa1 = jnp.zeros((UNITS, 128), jnp.float32).at[:, 0].set(W_alpha[:UNITS, 0])
    gbp = jnp.zeros((8, 3 * UNITS), jnp.float32).at[:2].set(gru_bias)

    wn_aug, h0 = pl.pallas_call(
        _prep_body,
        grid=(NB,),
        in_specs=[
            pl.BlockSpec((BLK, F), lambda i: (i, 0)),
            pl.BlockSpec((F, AW), lambda i: (0, 0)),
            pl.BlockSpec((1, AW), lambda i: (0, 0)),
            pl.BlockSpec((1, 1, BLK), lambda i: (i, 0, 0)),
        ],
        out_specs=[
            pl.BlockSpec((BLK, AW), lambda i: (i, 0)),
            pl.BlockSpec((BATCH, F), lambda i: (0, 0)),
        ],
        out_shape=[
            jax.ShapeDtypeStruct((NPAD, AW), jnp.float32),
            jax.ShapeDtypeStruct((BATCH, F), jnp.float32),
        ],
    )(nodep, W_aug, b_aug, bi3)

    sn_row = wn_aug[:, UNITS + 1].reshape(NB, 1, BLK)

    sh_col = pl.pallas_call(
        _sh_body,
        out_shape=jax.ShapeDtypeStruct((BATCH, 128), jnp.float32),
    )(h0, wa1)

    h = h0
    for _ in range(DEPTH):
        sh_row = jnp.broadcast_to(sh_col[:, 0].reshape(1, BATCH), (8, BATCH))
        acc = pl.pallas_call(
            _iter_body,
            grid=(NB,),
            in_specs=[
                pl.BlockSpec((BLK, AW), lambda i: (i, 0)),
                pl.BlockSpec((1, 1, BLK), lambda i: (i, 0, 0)),
                pl.BlockSpec((1, 1, BLK), lambda i: (i, 0, 0)),
                pl.BlockSpec((8, BATCH), lambda i: (0, 0)),
            ],
            out_specs=pl.BlockSpec((BATCH, AW), lambda i: (0, 0)),
            out_shape=jax.ShapeDtypeStruct((BATCH, AW), jnp.float32),
        )(wn_aug, sn_row, bi3, sh_row)
        h, sh_col = pl.pallas_call(
            _gru_body,
            out_shape=[
                jax.ShapeDtypeStruct((BATCH, UNITS), jnp.float32),
                jax.ShapeDtypeStruct((BATCH, 128), jnp.float32),
            ],
        )(acc, h, gru_kernel, gru_rec, gbp, wa1)
    return h


# speedup vs baseline: 6.4792x; 1.6146x over previous
import jax
import jax.numpy as jnp
from jax.experimental import pallas as pl

UNITS = 512
F = 512
BATCH = 256
DEPTH = 3
BLK = 512
AW = 640
_HI = jax.lax.Precision.HIGHEST


def _prep_body(node_ref, W_ref, b_ref, wa2_ref, ba_ref, bi_ref,
               wn_ref, sn_ref, h0_ref):
    i = pl.program_id(0)
    x = node_ref[...]
    wn_ref[...] = jnp.dot(x, W_ref[...], preferred_element_type=jnp.float32) + b_ref[...]
    sn_ref[...] = jnp.dot(x, wa2_ref[...], preferred_element_type=jnp.float32,
                          precision=_HI) + ba_ref[...]
    bi = bi_ref[0]
    oh = (jax.lax.broadcasted_iota(jnp.int32, (BATCH, BLK), 0) == bi).astype(jnp.float32)
    part = jnp.dot(oh, x, preferred_element_type=jnp.float32, precision=_HI)

    @pl.when(i == 0)
    def _():
        h0_ref[...] = part

    @pl.when(i > 0)
    def _():
        h0_ref[...] += part


def _sh_body(h_ref, wa1_ref, out_ref):
    out_ref[...] = jnp.dot(h_ref[...], wa1_ref[...],
                           preferred_element_type=jnp.float32, precision=_HI)


def _iter_body(wn_ref, sn_ref, bi_ref, sh_ref, acc_ref):
    i = pl.program_id(0)
    bi = bi_ref[0]
    oh = (jax.lax.broadcasted_iota(jnp.int32, (BATCH, BLK), 0) == bi).astype(jnp.float32)
    shg = jnp.dot(sh_ref[...], oh, preferred_element_type=jnp.float32,
                  precision=_HI)
    av = shg[0:1] + sn_ref[0]
    av = jnp.where(av > 0, av, 0.2 * av)
    e = jnp.exp(av)
    ow = oh * e
    part = jnp.dot(ow, wn_ref[...], preferred_element_type=jnp.float32)

    @pl.when(i == 0)
    def _():
        acc_ref[...] = part

    @pl.when(i > 0)
    def _():
        acc_ref[...] += part


def _gru_body(acc_ref, h_ref, gk_ref, gr_ref, gb_ref, wa1_ref, hn_ref, shc_ref):
    acc = acc_ref[...]
    denom = jnp.maximum(acc[:, UNITS:UNITS + 1], 1e-30)
    cont = acc[:, :UNITS] / denom
    cont = jnp.where(cont > 0, cont, jnp.exp(cont) - 1.0)
    h = h_ref[...]
    mx = jnp.dot(cont, gk_ref[...], preferred_element_type=jnp.float32) + gb_ref[0:1]
    mi = jnp.dot(h, gr_ref[...], preferred_element_type=jnp.float32) + gb_ref[1:2]
    xz, xr, xh = mx[:, :UNITS], mx[:, UNITS:2 * UNITS], mx[:, 2 * UNITS:]
    rz, rr, rh = mi[:, :UNITS], mi[:, UNITS:2 * UNITS], mi[:, 2 * UNITS:]
    z = jax.nn.sigmoid(xz + rz)
    r = jax.nn.sigmoid(xr + rr)
    hh = jnp.tanh(xh + r * rh)
    hn = z * h + (1.0 - z) * hh
    hn_ref[...] = hn
    shc_ref[...] = jnp.dot(hn, wa1_ref[...], preferred_element_type=jnp.float32,
                           precision=_HI)


def kernel(ref, node, batch_index, W_lin, b_lin, W_alpha, b_alpha,
           gru_kernel, gru_rec, gru_bias):
    N = node.shape[0]
    NB = -(-N // BLK)
    NPAD = NB * BLK
    nodep = jnp.pad(node, ((0, NPAD - N), (0, 0)))
    bip = jnp.pad(batch_index.astype(jnp.int32), (0, NPAD - N),
                  constant_values=BATCH)
    bi3 = bip.reshape(NB, 1, BLK)

    W_aug = jnp.zeros((F, AW), jnp.float32).at[:, :UNITS].set(W_lin)
    b_aug = (jnp.zeros((AW,), jnp.float32)
             .at[:UNITS].set(b_lin)
             .at[UNITS].set(1.0)).reshape(1, AW)
    wa2 = jnp.zeros((F, 128), jnp.float32).at[:, 0].set(W_alpha[UNITS:, 0])
    ba_row = jnp.broadcast_to(b_alpha.reshape(1, 1), (1, 128))
    wa1 = jnp.zeros((UNITS, 128), jnp.float32).at[:, 0].set(W_alpha[:UNITS, 0])
    gbp = jnp.zeros((8, 3 * UNITS), jnp.float32).at[:2].set(gru_bias)

    wn_aug, sn_col, h0 = pl.pallas_call(
        _prep_body,
        grid=(NB,),
        in_specs=[
            pl.BlockSpec((BLK, F), lambda i: (i, 0)),
            pl.BlockSpec((F, AW), lambda i: (0, 0)),
            pl.BlockSpec((1, AW), lambda i: (0, 0)),
            pl.BlockSpec((F, 128), lambda i: (0, 0)),
            pl.BlockSpec((1, 128), lambda i: (0, 0)),
            pl.BlockSpec((1, 1, BLK), lambda i: (i, 0, 0)),
        ],
        out_specs=[
            pl.BlockSpec((BLK, AW), lambda i: (i, 0)),
            pl.BlockSpec((BLK, 128), lambda i: (i, 0)),
            pl.BlockSpec((BATCH, F), lambda i: (0, 0)),
        ],
        out_shape=[
            jax.ShapeDtypeStruct((NPAD, AW), jnp.float32),
            jax.ShapeDtypeStruct((NPAD, 128), jnp.float32),
            jax.ShapeDtypeStruct((BATCH, F), jnp.float32),
        ],
    )(nodep, W_aug, b_aug, wa2, ba_row, bi3)

    sn_row = sn_col[:, 0].reshape(NB, 1, BLK)

    sh_col = pl.pallas_call(
        _sh_body,
        out_shape=jax.ShapeDtypeStruct((BATCH, 128), jnp.float32),
    )(h0, wa1)

    h = h0
    for _ in range(DEPTH):
        sh_row = jnp.broadcast_to(sh_col[:, 0].reshape(1, BATCH), (8, BATCH))
        acc = pl.pallas_call(
            _iter_body,
            grid=(NB,),
            in_specs=[
                pl.BlockSpec((BLK, AW), lambda i: (i, 0)),
                pl.BlockSpec((1, 1, BLK), lambda i: (i, 0, 0)),
                pl.BlockSpec((1, 1, BLK), lambda i: (i, 0, 0)),
                pl.BlockSpec((8, BATCH), lambda i: (0, 0)),
            ],
            out_specs=pl.BlockSpec((BATCH, AW), lambda i: (0, 0)),
            out_shape=jax.ShapeDtypeStruct((BATCH, AW), jnp.float32),
        )(wn_aug, sn_row, bi3, sh_row)
        h, sh_col = pl.pallas_call(
            _gru_body,
            out_shape=[
                jax.ShapeDtypeStruct((BATCH, UNITS), jnp.float32),
                jax.ShapeDtypeStruct((BATCH, 128), jnp.float32),
            ],
        )(acc, h, gru_kernel, gru_rec, gbp, wa1)
    return h


# speedup vs baseline: 8.4043x; 1.2971x over previous
import jax
import jax.numpy as jnp
from jax.experimental import pallas as pl
from jax.experimental.pallas import tpu as pltpu

UNITS = 512
F = 512
BATCH = 256
DEPTH = 3
BLK = 512
AW = 640
_HI = jax.lax.Precision.HIGHEST


def _f32(a, b):
    return jnp.dot(a, b, preferred_element_type=jnp.float32)


def _split(x):
    hi = x.astype(jnp.bfloat16)
    lo = (x - hi.astype(jnp.float32)).astype(jnp.bfloat16)
    return hi, lo


def _prep_body(node_ref, W_ref, b_ref, w2h_ref, w2l_ref, ba_ref, wa1_ref,
               bi_ref, wn_ref, sn_ref, h0_ref, sh0_ref, acc_ref):
    i = pl.program_id(0)
    nb = pl.num_programs(0)
    x = node_ref[...]
    wn = _f32(x, W_ref[...]) + b_ref[...]
    wn_ref[...] = wn.astype(jnp.bfloat16)
    xh, xl = _split(x)
    sn_ref[...] = (_f32(xh, w2h_ref[...]) + _f32(xh, w2l_ref[...])
                   + _f32(xl, w2h_ref[...])) + ba_ref[...]
    bi = bi_ref[0]
    oh = (jax.lax.broadcasted_iota(jnp.int32, (BATCH, BLK), 0) == bi
          ).astype(jnp.bfloat16)
    part = _f32(oh, xh) + _f32(oh, xl)

    @pl.when(i == 0)
    def _():
        acc_ref[...] = part

    @pl.when(i > 0)
    def _():
        acc_ref[...] += part

    @pl.when(i == nb - 1)
    def _():
        h0 = acc_ref[...]
        h0_ref[...] = h0
        sh0_ref[...] = jnp.dot(h0, wa1_ref[...],
                               preferred_element_type=jnp.float32,
                               precision=_HI)


def _iter_body(wn_ref, sn_ref, bi_ref, shh_ref, shl_ref, h_ref,
               gk_ref, gr_ref, gb_ref, wa1_ref, hn_ref, shc_ref, acc_ref):
    i = pl.program_id(0)
    nb = pl.num_programs(0)
    bi = bi_ref[0]
    oh = (jax.lax.broadcasted_iota(jnp.int32, (BATCH, BLK), 0) == bi
          ).astype(jnp.bfloat16)
    shg = _f32(shh_ref[...], oh) + _f32(shl_ref[...], oh)
    av = shg[0:1] + sn_ref[0]
    av = jnp.where(av > 0, av, 0.2 * av)
    e = jnp.exp(av)
    ow = (oh.astype(jnp.float32) * e).astype(jnp.bfloat16)
    part = _f32(ow, wn_ref[...])

    @pl.when(i == 0)
    def _():
        acc_ref[...] = part

    @pl.when(i > 0)
    def _():
        acc_ref[...] += part

    @pl.when(i == nb - 1)
    def _():
        acc = acc_ref[...]
        denom = jnp.maximum(acc[:, UNITS:UNITS + 1], 1e-30)
        cont = acc[:, :UNITS] / denom
        cont = jnp.where(cont > 0, cont, jnp.exp(cont) - 1.0)
        h = h_ref[...]
        mx = _f32(cont, gk_ref[...]) + gb_ref[0:1]
        mi = _f32(h, gr_ref[...]) + gb_ref[1:2]
        xz, xr, xg = mx[:, :UNITS], mx[:, UNITS:2 * UNITS], mx[:, 2 * UNITS:]
        rz, rr, rg = mi[:, :UNITS], mi[:, UNITS:2 * UNITS], mi[:, 2 * UNITS:]
        z = jax.nn.sigmoid(xz + rz)
        r = jax.nn.sigmoid(xr + rr)
        hh = jnp.tanh(xg + r * rg)
        hn = z * h + (1.0 - z) * hh
        hn_ref[...] = hn
        shc_ref[...] = jnp.dot(hn, wa1_ref[...],
                               preferred_element_type=jnp.float32,
                               precision=_HI)


def kernel(ref, node, batch_index, W_lin, b_lin, W_alpha, b_alpha,
           gru_kernel, gru_rec, gru_bias):
    N = node.shape[0]
    NB = -(-N // BLK)
    NPAD = NB * BLK
    nodep = jnp.pad(node, ((0, NPAD - N), (0, 0)))
    bip = jnp.pad(batch_index.astype(jnp.int32), (0, NPAD - N),
                  constant_values=BATCH)
    bi3 = bip.reshape(NB, 1, BLK)

    W_aug = jnp.zeros((F, AW), jnp.float32).at[:, :UNITS].set(W_lin)
    b_aug = (jnp.zeros((AW,), jnp.float32)
             .at[:UNITS].set(b_lin)
             .at[UNITS].set(1.0)).reshape(1, AW)
    wa2 = jnp.zeros((F, 128), jnp.float32).at[:, 0].set(W_alpha[UNITS:, 0])
    w2h = wa2.astype(jnp.bfloat16)
    w2l = (wa2 - w2h.astype(jnp.float32)).astype(jnp.bfloat16)
    ba_row = jnp.broadcast_to(b_alpha.reshape(1, 1), (1, 128))
    wa1 = jnp.zeros((UNITS, 128), jnp.float32).at[:, 0].set(W_alpha[:UNITS, 0])
    gbp = jnp.zeros((8, 3 * UNITS), jnp.float32).at[:2].set(gru_bias)

    def full(shape):
        nzero = len(shape)
        return pl.BlockSpec(shape, lambda *args, _n=nzero: (0,) * _n)

    wn_aug, sn_col, h0, sh_col = pl.pallas_call(
        _prep_body,
        grid=(NB,),
        in_specs=[
            pl.BlockSpec((BLK, F), lambda i: (i, 0)),
            full((F, AW)),
            full((1, AW)),
            full((F, 128)),
            full((F, 128)),
            full((1, 128)),
            full((UNITS, 128)),
            pl.BlockSpec((1, 1, BLK), lambda i: (i, 0, 0)),
        ],
        out_specs=[
            pl.BlockSpec((BLK, AW), lambda i: (i, 0)),
            pl.BlockSpec((BLK, 128), lambda i: (i, 0)),
            full((BATCH, F)),
            full((BATCH, 128)),
        ],
        out_shape=[
            jax.ShapeDtypeStruct((NPAD, AW), jnp.bfloat16),
            jax.ShapeDtypeStruct((NPAD, 128), jnp.float32),
            jax.ShapeDtypeStruct((BATCH, F), jnp.float32),
            jax.ShapeDtypeStruct((BATCH, 128), jnp.float32),
        ],
        scratch_shapes=[pltpu.VMEM((BATCH, F), jnp.float32)],
    )(nodep, W_aug, b_aug, w2h, w2l, ba_row, wa1, bi3)

    sn_row = sn_col[:, 0].reshape(NB, 1, BLK)

    h = h0
    for _ in range(DEPTH):
        sh8 = jnp.broadcast_to(sh_col[:, 0].reshape(1, BATCH), (8, BATCH))
        shh = sh8.astype(jnp.bfloat16)
        shl = (sh8 - shh.astype(jnp.float32)).astype(jnp.bfloat16)
        h, sh_col = pl.pallas_call(
            _iter_body,
            grid=(NB,),
            in_specs=[
                pl.BlockSpec((BLK, AW), lambda i: (i, 0)),
                pl.BlockSpec((1, 1, BLK), lambda i: (i, 0, 0)),
                pl.BlockSpec((1, 1, BLK), lambda i: (i, 0, 0)),
                full((8, BATCH)),
                full((8, BATCH)),
                full((BATCH, UNITS)),
                full((UNITS, 3 * UNITS)),
                full((UNITS, 3 * UNITS)),
                full((8, 3 * UNITS)),
                full((UNITS, 128)),
            ],
            out_specs=[
                full((BATCH, UNITS)),
                full((BATCH, 128)),
            ],
            out_shape=[
                jax.ShapeDtypeStruct((BATCH, UNITS), jnp.float32),
                jax.ShapeDtypeStruct((BATCH, 128), jnp.float32),
            ],
            scratch_shapes=[pltpu.VMEM((BATCH, AW), jnp.float32)],
        )(wn_aug, sn_row, bi3, shh, shl, h, gru_kernel, gru_rec, gbp, wa1)
    return h


# speedup vs baseline: 8.6359x; 1.0276x over previous
import jax
import jax.numpy as jnp
from jax.experimental import pallas as pl
from jax.experimental.pallas import tpu as pltpu

UNITS = 512
F = 512
BATCH = 256
DEPTH = 3
BLK = 512
AW = 640
_HI = jax.lax.Precision.HIGHEST


def _f32(a, b):
    return jnp.dot(a, b, preferred_element_type=jnp.float32)


def _split(x):
    hi = x.astype(jnp.bfloat16)
    lo = (x - hi.astype(jnp.float32)).astype(jnp.bfloat16)
    return hi, lo


def _prep_body(node_ref, W_ref, b_ref, w2h_ref, w2l_ref, ba_ref, wa1_ref,
               bi_ref, wn_ref, sn_ref, h0_ref, sh0_ref, acc_ref):
    i = pl.program_id(0)
    nb = pl.num_programs(0)
    x = node_ref[...]
    wn = _f32(x, W_ref[...]) + b_ref[...]
    wn_ref[...] = wn.astype(jnp.bfloat16)
    xh, xl = _split(x)
    sn_ref[...] = (_f32(xh, w2h_ref[...]) + _f32(xh, w2l_ref[...])
                   + _f32(xl, w2h_ref[...])) + ba_ref[...]
    bi = bi_ref[0]
    oh = (jax.lax.broadcasted_iota(jnp.int32, (BATCH, BLK), 0) == bi
          ).astype(jnp.bfloat16)
    part = _f32(oh, xh) + _f32(oh, xl)

    @pl.when(i == 0)
    def _():
        acc_ref[...] = part

    @pl.when(i > 0)
    def _():
        acc_ref[...] += part

    @pl.when(i == nb - 1)
    def _():
        h0 = acc_ref[...]
        h0_ref[...] = h0
        sh0_ref[...] = jnp.dot(h0, wa1_ref[...],
                               preferred_element_type=jnp.float32,
                               precision=_HI)


def _loop_body(wn_ref, sn_ref, bi_ref, sh0_ref, h0_ref,
               gk_ref, gr_ref, gb_ref, wa1_ref, hn_ref,
               acc_ref, h_scr, sh_scr):
    d = pl.program_id(0)
    i = pl.program_id(1)
    nb = pl.num_programs(1)

    @pl.when((d == 0) & (i == 0))
    def _():
        h_scr[...] = h0_ref[...]
        sh_scr[...] = sh0_ref[...]

    bi = bi_ref[0]
    mask = jax.lax.broadcasted_iota(jnp.int32, (BATCH, BLK), 0) == bi
    oh = mask.astype(jnp.bfloat16)
    sh = sh_scr[...]
    shh = sh.astype(jnp.bfloat16)
    shl = (sh - shh.astype(jnp.float32)).astype(jnp.bfloat16)
    shg = _f32(shh, oh) + _f32(shl, oh)
    av = shg[0:1] + sn_ref[0]
    av = jnp.where(av > 0, av, 0.2 * av)
    e = jnp.exp(av)
    ow = jnp.where(mask, jnp.broadcast_to(e, (BATCH, BLK)),
                   0.0).astype(jnp.bfloat16)
    part = _f32(ow, wn_ref[...])

    @pl.when(i == 0)
    def _():
        acc_ref[...] = part

    @pl.when(i > 0)
    def _():
        acc_ref[...] += part

    @pl.when(i == nb - 1)
    def _():
        acc = acc_ref[...]
        denom = jnp.maximum(acc[:, UNITS:UNITS + 1], 1e-30)
        cont = acc[:, :UNITS] / denom
        cont = jnp.where(cont > 0, cont, jnp.exp(cont) - 1.0)
        h = h_scr[...]
        mx = _f32(cont, gk_ref[...]) + gb_ref[0:1]
        mi = _f32(h, gr_ref[...]) + gb_ref[1:2]
        xz, xr, xg = mx[:, :UNITS], mx[:, UNITS:2 * UNITS], mx[:, 2 * UNITS:]
        rz, rr, rg = mi[:, :UNITS], mi[:, UNITS:2 * UNITS], mi[:, 2 * UNITS:]
        z = jax.nn.sigmoid(xz + rz)
        r = jax.nn.sigmoid(xr + rr)
        hh = jnp.tanh(xg + r * rg)
        hn = z * h + (1.0 - z) * hh
        h_scr[...] = hn
        shc = jnp.dot(hn, wa1_ref[...], preferred_element_type=jnp.float32,
                      precision=_HI)
        sh_scr[...] = jnp.broadcast_to(shc[:, 0].reshape(1, BATCH), (8, BATCH))

        @pl.when(d == DEPTH - 1)
        def _():
            hn_ref[...] = hn


def kernel(ref, node, batch_index, W_lin, b_lin, W_alpha, b_alpha,
           gru_kernel, gru_rec, gru_bias):
    N = node.shape[0]
    NB = -(-N // BLK)
    NPAD = NB * BLK
    nodep = jnp.pad(node, ((0, NPAD - N), (0, 0)))
    bip = jnp.pad(batch_index.astype(jnp.int32), (0, NPAD - N),
                  constant_values=BATCH)
    bi3 = bip.reshape(NB, 1, BLK)

    W_aug = jnp.zeros((F, AW), jnp.float32).at[:, :UNITS].set(W_lin)
    b_aug = (jnp.zeros((AW,), jnp.float32)
             .at[:UNITS].set(b_lin)
             .at[UNITS].set(1.0)).reshape(1, AW)
    wa2 = jnp.zeros((F, 128), jnp.float32).at[:, 0].set(W_alpha[UNITS:, 0])
    w2h = wa2.astype(jnp.bfloat16)
    w2l = (wa2 - w2h.astype(jnp.float32)).astype(jnp.bfloat16)
    ba_row = jnp.broadcast_to(b_alpha.reshape(1, 1), (1, 128))
    wa1 = jnp.zeros((UNITS, 128), jnp.float32).at[:, 0].set(W_alpha[:UNITS, 0])
    gbp = jnp.zeros((8, 3 * UNITS), jnp.float32).at[:2].set(gru_bias)

    def full(shape):
        nzero = len(shape)
        return pl.BlockSpec(shape, lambda *args, _n=nzero: (0,) * _n)

    wn_aug, sn_col, h0, sh_col = pl.pallas_call(
        _prep_body,
        grid=(NB,),
        in_specs=[
            pl.BlockSpec((BLK, F), lambda i: (i, 0)),
            full((F, AW)),
            full((1, AW)),
            full((F, 128)),
            full((F, 128)),
            full((1, 128)),
            full((UNITS, 128)),
            pl.BlockSpec((1, 1, BLK), lambda i: (i, 0, 0)),
        ],
        out_specs=[
            pl.BlockSpec((BLK, AW), lambda i: (i, 0)),
            pl.BlockSpec((BLK, 128), lambda i: (i, 0)),
            full((BATCH, F)),
            full((BATCH, 128)),
        ],
        out_shape=[
            jax.ShapeDtypeStruct((NPAD, AW), jnp.bfloat16),
            jax.ShapeDtypeStruct((NPAD, 128), jnp.float32),
            jax.ShapeDtypeStruct((BATCH, F), jnp.float32),
            jax.ShapeDtypeStruct((BATCH, 128), jnp.float32),
        ],
        scratch_shapes=[pltpu.VMEM((BATCH, F), jnp.float32)],
    )(nodep, W_aug, b_aug, w2h, w2l, ba_row, wa1, bi3)

    sn_row = sn_col[:, 0].reshape(NB, 1, BLK)
    sh08 = jnp.broadcast_to(sh_col[:, 0].reshape(1, BATCH), (8, BATCH))

    h = pl.pallas_call(
        _loop_body,
        grid=(DEPTH, NB),
        in_specs=[
            pl.BlockSpec((BLK, AW), lambda d, i: (i, 0)),
            pl.BlockSpec((1, 1, BLK), lambda d, i: (i, 0, 0)),
            pl.BlockSpec((1, 1, BLK), lambda d, i: (i, 0, 0)),
            full((8, BATCH)),
            full((BATCH, UNITS)),
            full((UNITS, 3 * UNITS)),
            full((UNITS, 3 * UNITS)),
            full((8, 3 * UNITS)),
            full((UNITS, 128)),
        ],
        out_specs=full((BATCH, UNITS)),
        out_shape=jax.ShapeDtypeStruct((BATCH, UNITS), jnp.float32),
        scratch_shapes=[
            pltpu.VMEM((BATCH, AW), jnp.float32),
            pltpu.VMEM((BATCH, UNITS), jnp.float32),
            pltpu.VMEM((8, BATCH), jnp.float32),
        ],
    )(wn_aug, sn_row, bi3, sh08, h0, gru_kernel, gru_rec, gbp, wa1)
    return h


# speedup vs baseline: 11.3123x; 1.3099x over previous
import jax
import jax.numpy as jnp
from jax.experimental import pallas as pl
from jax.experimental.pallas import tpu as pltpu

UNITS = 512
F = 512
BATCH = 256
DEPTH = 3
BLK = 1024
AW = 640
_HI = jax.lax.Precision.HIGHEST


def _f32(a, b):
    return jnp.dot(a, b, preferred_element_type=jnp.float32)


def _split(x):
    hi = x.astype(jnp.bfloat16)
    lo = (x - hi.astype(jnp.float32)).astype(jnp.bfloat16)
    return hi, lo


def _prep_body(node_ref, W_ref, b_ref, w2h_ref, w2l_ref, ba_ref, wa1_ref,
               bi_ref, wn_ref, sn_ref, h0_ref, sh0_ref, acc_ref):
    i = pl.program_id(0)
    nb = pl.num_programs(0)
    x = node_ref[...]
    wn = _f32(x, W_ref[...]) + b_ref[...]
    wn_ref[...] = wn.astype(jnp.bfloat16)
    xh, xl = _split(x)
    sn_ref[...] = (_f32(xh, w2h_ref[...]) + _f32(xh, w2l_ref[...])
                   + _f32(xl, w2h_ref[...])) + ba_ref[...]
    bi = bi_ref[0]
    oh = (jax.lax.broadcasted_iota(jnp.int32, (BATCH, BLK), 0) == bi
          ).astype(jnp.bfloat16)
    part = _f32(oh, xh) + _f32(oh, xl)

    @pl.when(i == 0)
    def _():
        acc_ref[...] = part

    @pl.when(i > 0)
    def _():
        acc_ref[...] += part

    @pl.when(i == nb - 1)
    def _():
        h0 = acc_ref[...]
        h0_ref[...] = h0
        sh0_ref[...] = jnp.dot(h0, wa1_ref[...],
                               preferred_element_type=jnp.float32,
                               precision=_HI)


def _loop_body(wn_ref, sn_ref, bi_ref, sh0_ref, h0_ref,
               gk_ref, gr_ref, gb_ref, wa1_ref, hn_ref,
               acc_ref, h_scr, sh_scr):
    d = pl.program_id(0)
    i = pl.program_id(1)
    nb = pl.num_programs(1)

    @pl.when((d == 0) & (i == 0))
    def _():
        h_scr[...] = h0_ref[...]
        sh_scr[...] = sh0_ref[...]

    bi = bi_ref[0]
    mask = jax.lax.broadcasted_iota(jnp.int32, (BATCH, BLK), 0) == bi
    oh = mask.astype(jnp.bfloat16)
    sh = sh_scr[...]
    shh = sh.astype(jnp.bfloat16)
    shl = (sh - shh.astype(jnp.float32)).astype(jnp.bfloat16)
    shg = _f32(shh, oh) + _f32(shl, oh)
    av = shg[0:1] + sn_ref[0]
    av = jnp.where(av > 0, av, 0.2 * av)
    e = jnp.exp(av)
    ow = jnp.where(mask, jnp.broadcast_to(e, (BATCH, BLK)),
                   0.0).astype(jnp.bfloat16)
    part = _f32(ow, wn_ref[...])

    @pl.when(i == 0)
    def _():
        acc_ref[...] = part

    @pl.when(i > 0)
    def _():
        acc_ref[...] += part

    @pl.when(i == nb - 1)
    def _():
        acc = acc_ref[...]
        denom = jnp.maximum(acc[:, UNITS:UNITS + 1], 1e-30)
        cont = acc[:, :UNITS] / denom
        cont = jnp.where(cont > 0, cont, jnp.exp(cont) - 1.0)
        h = h_scr[...]
        mx = _f32(cont, gk_ref[...]) + gb_ref[0:1]
        mi = _f32(h, gr_ref[...]) + gb_ref[1:2]
        xz, xr, xg = mx[:, :UNITS], mx[:, UNITS:2 * UNITS], mx[:, 2 * UNITS:]
        rz, rr, rg = mi[:, :UNITS], mi[:, UNITS:2 * UNITS], mi[:, 2 * UNITS:]
        z = jax.nn.sigmoid(xz + rz)
        r = jax.nn.sigmoid(xr + rr)
        hh = jnp.tanh(xg + r * rg)
        hn = z * h + (1.0 - z) * hh
        h_scr[...] = hn
        shc = jnp.dot(hn, wa1_ref[...], preferred_element_type=jnp.float32,
                      precision=_HI)
        sh_scr[...] = jnp.broadcast_to(shc[:, 0].reshape(1, BATCH), (8, BATCH))

        @pl.when(d == DEPTH - 1)
        def _():
            hn_ref[...] = hn


def kernel(ref, node, batch_index, W_lin, b_lin, W_alpha, b_alpha,
           gru_kernel, gru_rec, gru_bias):
    N = node.shape[0]
    NB = -(-N // BLK)
    NPAD = NB * BLK
    nodep = jnp.pad(node, ((0, NPAD - N), (0, 0)))
    bip = jnp.pad(batch_index.astype(jnp.int32), (0, NPAD - N),
                  constant_values=BATCH)
    bi3 = bip.reshape(NB, 1, BLK)

    W_aug = jnp.zeros((F, AW), jnp.float32).at[:, :UNITS].set(W_lin)
    b_aug = (jnp.zeros((AW,), jnp.float32)
             .at[:UNITS].set(b_lin)
             .at[UNITS].set(1.0)).reshape(1, AW)
    wa2 = jnp.zeros((F, 128), jnp.float32).at[:, 0].set(W_alpha[UNITS:, 0])
    w2h = wa2.astype(jnp.bfloat16)
    w2l = (wa2 - w2h.astype(jnp.float32)).astype(jnp.bfloat16)
    ba_row = jnp.broadcast_to(b_alpha.reshape(1, 1), (1, 128))
    wa1 = jnp.zeros((UNITS, 128), jnp.float32).at[:, 0].set(W_alpha[:UNITS, 0])
    gbp = jnp.zeros((8, 3 * UNITS), jnp.float32).at[:2].set(gru_bias)

    def full(shape):
        nzero = len(shape)
        return pl.BlockSpec(shape, lambda *args, _n=nzero: (0,) * _n)

    wn_aug, sn_col, h0, sh_col = pl.pallas_call(
        _prep_body,
        grid=(NB,),
        in_specs=[
            pl.BlockSpec((BLK, F), lambda i: (i, 0)),
            full((F, AW)),
            full((1, AW)),
            full((F, 128)),
            full((F, 128)),
            full((1, 128)),
            full((UNITS, 128)),
            pl.BlockSpec((1, 1, BLK), lambda i: (i, 0, 0)),
        ],
        out_specs=[
            pl.BlockSpec((BLK, AW), lambda i: (i, 0)),
            pl.BlockSpec((BLK, 128), lambda i: (i, 0)),
            full((BATCH, F)),
            full((BATCH, 128)),
        ],
        out_shape=[
            jax.ShapeDtypeStruct((NPAD, AW), jnp.bfloat16),
            jax.ShapeDtypeStruct((NPAD, 128), jnp.float32),
            jax.ShapeDtypeStruct((BATCH, F), jnp.float32),
            jax.ShapeDtypeStruct((BATCH, 128), jnp.float32),
        ],
        scratch_shapes=[pltpu.VMEM((BATCH, F), jnp.float32)],
    )(nodep, W_aug, b_aug, w2h, w2l, ba_row, wa1, bi3)

    sn_row = sn_col[:, 0].reshape(NB, 1, BLK)
    sh08 = jnp.broadcast_to(sh_col[:, 0].reshape(1, BATCH), (8, BATCH))

    h = pl.pallas_call(
        _loop_body,
        grid=(DEPTH, NB),
        in_specs=[
            pl.BlockSpec((BLK, AW), lambda d, i: (i, 0)),
            pl.BlockSpec((1, 1, BLK), lambda d, i: (i, 0, 0)),
            pl.BlockSpec((1, 1, BLK), lambda d, i: (i, 0, 0)),
            full((8, BATCH)),
            full((BATCH, UNITS)),
            full((UNITS, 3 * UNITS)),
            full((UNITS, 3 * UNITS)),
            full((8, 3 * UNITS)),
            full((UNITS, 128)),
        ],
        out_specs=full((BATCH, UNITS)),
        out_shape=jax.ShapeDtypeStruct((BATCH, UNITS), jnp.float32),
        scratch_shapes=[
            pltpu.VMEM((BATCH, AW), jnp.float32),
            pltpu.VMEM((BATCH, UNITS), jnp.float32),
            pltpu.VMEM((8, BATCH), jnp.float32),
        ],
    )(wn_aug, sn_row, bi3, sh08, h0, gru_kernel, gru_rec, gbp, wa1)
    return h


# speedup vs baseline: 13.0561x; 1.1541x over previous
import jax
import jax.numpy as jnp
from jax.experimental import pallas as pl
from jax.experimental.pallas import tpu as pltpu

UNITS = 512
F = 512
BATCH = 256
DEPTH = 3
BLK = 2048
AW = 640
_HI = jax.lax.Precision.HIGHEST


def _f32(a, b):
    return jnp.dot(a, b, preferred_element_type=jnp.float32)


def _split(x):
    hi = x.astype(jnp.bfloat16)
    lo = (x - hi.astype(jnp.float32)).astype(jnp.bfloat16)
    return hi, lo


def _prep_body(node_ref, W_ref, b_ref, w2h_ref, w2l_ref, ba_ref, wa1_ref,
               bi_ref, wn_ref, sn_ref, h0_ref, sh0_ref, acc_ref):
    i = pl.program_id(0)
    nb = pl.num_programs(0)
    x = node_ref[...]
    wn = _f32(x, W_ref[...]) + b_ref[...]
    wn_ref[...] = wn.astype(jnp.bfloat16)
    xh, xl = _split(x)
    sn_ref[...] = (_f32(xh, w2h_ref[...]) + _f32(xh, w2l_ref[...])
                   + _f32(xl, w2h_ref[...])) + ba_ref[...]
    bi = bi_ref[0]
    oh = (jax.lax.broadcasted_iota(jnp.int32, (BATCH, BLK), 0) == bi
          ).astype(jnp.bfloat16)
    part = _f32(oh, xh) + _f32(oh, xl)

    @pl.when(i == 0)
    def _():
        acc_ref[...] = part

    @pl.when(i > 0)
    def _():
        acc_ref[...] += part

    @pl.when(i == nb - 1)
    def _():
        h0 = acc_ref[...]
        h0_ref[...] = h0
        sh0_ref[...] = jnp.dot(h0, wa1_ref[...],
                               preferred_element_type=jnp.float32,
                               precision=_HI)


def _loop_body(wn_ref, sn_ref, bi_ref, sh0_ref, h0_ref,
               gk_ref, gr_ref, gb_ref, wa1_ref, hn_ref,
               acc_ref, h_scr, sh_scr):
    d = pl.program_id(0)
    i = pl.program_id(1)
    nb = pl.num_programs(1)

    @pl.when((d == 0) & (i == 0))
    def _():
        h_scr[...] = h0_ref[...]
        sh_scr[...] = sh0_ref[...]

    bi = bi_ref[0]
    mask = jax.lax.broadcasted_iota(jnp.int32, (BATCH, BLK), 0) == bi
    oh = mask.astype(jnp.bfloat16)
    sh = sh_scr[...]
    shh = sh.astype(jnp.bfloat16)
    shl = (sh - shh.astype(jnp.float32)).astype(jnp.bfloat16)
    shg = _f32(shh, oh) + _f32(shl, oh)
    av = shg[0:1] + sn_ref[0]
    av = jnp.where(av > 0, av, 0.2 * av)
    e = jnp.exp(av)
    ow = jnp.where(mask, jnp.broadcast_to(e, (BATCH, BLK)),
                   0.0).astype(jnp.bfloat16)
    part = _f32(ow, wn_ref[...])

    @pl.when(i == 0)
    def _():
        acc_ref[...] = part

    @pl.when(i > 0)
    def _():
        acc_ref[...] += part

    @pl.when(i == nb - 1)
    def _():
        acc = acc_ref[...]
        denom = jnp.maximum(acc[:, UNITS:UNITS + 1], 1e-30)
        cont = acc[:, :UNITS] / denom
        cont = jnp.where(cont > 0, cont, jnp.exp(cont) - 1.0)
        h = h_scr[...]
        mx = _f32(cont, gk_ref[...]) + gb_ref[0:1]
        mi = _f32(h, gr_ref[...]) + gb_ref[1:2]
        xz, xr, xg = mx[:, :UNITS], mx[:, UNITS:2 * UNITS], mx[:, 2 * UNITS:]
        rz, rr, rg = mi[:, :UNITS], mi[:, UNITS:2 * UNITS], mi[:, 2 * UNITS:]
        z = jax.nn.sigmoid(xz + rz)
        r = jax.nn.sigmoid(xr + rr)
        hh = jnp.tanh(xg + r * rg)
        hn = z * h + (1.0 - z) * hh
        h_scr[...] = hn
        shc = jnp.dot(hn, wa1_ref[...], preferred_element_type=jnp.float32,
                      precision=_HI)
        sh_scr[...] = jnp.broadcast_to(shc[:, 0].reshape(1, BATCH), (8, BATCH))

        @pl.when(d == DEPTH - 1)
        def _():
            hn_ref[...] = hn


def kernel(ref, node, batch_index, W_lin, b_lin, W_alpha, b_alpha,
           gru_kernel, gru_rec, gru_bias):
    N = node.shape[0]
    NB = -(-N // BLK)
    NPAD = NB * BLK
    nodep = jnp.pad(node, ((0, NPAD - N), (0, 0)))
    bip = jnp.pad(batch_index.astype(jnp.int32), (0, NPAD - N),
                  constant_values=BATCH)
    bi3 = bip.reshape(NB, 1, BLK)

    W_aug = jnp.zeros((F, AW), jnp.float32).at[:, :UNITS].set(W_lin)
    b_aug = (jnp.zeros((AW,), jnp.float32)
             .at[:UNITS].set(b_lin)
             .at[UNITS].set(1.0)).reshape(1, AW)
    wa2 = jnp.zeros((F, 128), jnp.float32).at[:, 0].set(W_alpha[UNITS:, 0])
    w2h = wa2.astype(jnp.bfloat16)
    w2l = (wa2 - w2h.astype(jnp.float32)).astype(jnp.bfloat16)
    ba_row = jnp.broadcast_to(b_alpha.reshape(1, 1), (1, 128))
    wa1 = jnp.zeros((UNITS, 128), jnp.float32).at[:, 0].set(W_alpha[:UNITS, 0])
    gbp = jnp.zeros((8, 3 * UNITS), jnp.float32).at[:2].set(gru_bias)

    def full(shape):
        nzero = len(shape)
        return pl.BlockSpec(shape, lambda *args, _n=nzero: (0,) * _n)

    wn_aug, sn_col, h0, sh_col = pl.pallas_call(
        _prep_body,
        grid=(NB,),
        in_specs=[
            pl.BlockSpec((BLK, F), lambda i: (i, 0)),
            full((F, AW)),
            full((1, AW)),
            full((F, 128)),
            full((F, 128)),
            full((1, 128)),
            full((UNITS, 128)),
            pl.BlockSpec((1, 1, BLK), lambda i: (i, 0, 0)),
        ],
        out_specs=[
            pl.BlockSpec((BLK, AW), lambda i: (i, 0)),
            pl.BlockSpec((BLK, 128), lambda i: (i, 0)),
            full((BATCH, F)),
            full((BATCH, 128)),
        ],
        out_shape=[
            jax.ShapeDtypeStruct((NPAD, AW), jnp.bfloat16),
            jax.ShapeDtypeStruct((NPAD, 128), jnp.float32),
            jax.ShapeDtypeStruct((BATCH, F), jnp.float32),
            jax.ShapeDtypeStruct((BATCH, 128), jnp.float32),
        ],
        scratch_shapes=[pltpu.VMEM((BATCH, F), jnp.float32)],
    )(nodep, W_aug, b_aug, w2h, w2l, ba_row, wa1, bi3)

    sn_row = sn_col[:, 0].reshape(NB, 1, BLK)
    sh08 = jnp.broadcast_to(sh_col[:, 0].reshape(1, BATCH), (8, BATCH))

    h = pl.pallas_call(
        _loop_body,
        grid=(DEPTH, NB),
        in_specs=[
            pl.BlockSpec((BLK, AW), lambda d, i: (i, 0)),
            pl.BlockSpec((1, 1, BLK), lambda d, i: (i, 0, 0)),
            pl.BlockSpec((1, 1, BLK), lambda d, i: (i, 0, 0)),
            full((8, BATCH)),
            full((BATCH, UNITS)),
            full((UNITS, 3 * UNITS)),
            full((UNITS, 3 * UNITS)),
            full((8, 3 * UNITS)),
            full((UNITS, 128)),
        ],
        out_specs=full((BATCH, UNITS)),
        out_shape=jax.ShapeDtypeStruct((BATCH, UNITS), jnp.float32),
        scratch_shapes=[
            pltpu.VMEM((BATCH, AW), jnp.float32),
            pltpu.VMEM((BATCH, UNITS), jnp.float32),
            pltpu.VMEM((8, BATCH), jnp.float32),
        ],
    )(wn_aug, sn_row, bi3, sh08, h0, gru_kernel, gru_rec, gbp, wa1)
    return h


# speedup vs baseline: 13.6655x; 1.0467x over previous
import jax
import jax.numpy as jnp
from jax.experimental import pallas as pl
from jax.experimental.pallas import tpu as pltpu

UNITS = 512
F = 512
BATCH = 256
DEPTH = 3
BLK = 4096
AW = 640
_HI = jax.lax.Precision.HIGHEST


def _f32(a, b):
    return jnp.dot(a, b, preferred_element_type=jnp.float32)


def _split(x):
    hi = x.astype(jnp.bfloat16)
    lo = (x - hi.astype(jnp.float32)).astype(jnp.bfloat16)
    return hi, lo


def _prep_body(node_ref, W_ref, b_ref, w2h_ref, w2l_ref, ba_ref, wa1_ref,
               bi_ref, wn_ref, sn_ref, h0_ref, sh0_ref, acc_ref):
    i = pl.program_id(0)
    nb = pl.num_programs(0)
    x = node_ref[...]
    wn = _f32(x, W_ref[...]) + b_ref[...]
    wn_ref[...] = wn.astype(jnp.bfloat16)
    xh, xl = _split(x)
    sn_ref[...] = (_f32(xh, w2h_ref[...]) + _f32(xh, w2l_ref[...])
                   + _f32(xl, w2h_ref[...])) + ba_ref[...]
    bi = bi_ref[0]
    oh = (jax.lax.broadcasted_iota(jnp.int32, (BATCH, BLK), 0) == bi
          ).astype(jnp.bfloat16)
    part = _f32(oh, xh) + _f32(oh, xl)

    @pl.when(i == 0)
    def _():
        acc_ref[...] = part

    @pl.when(i > 0)
    def _():
        acc_ref[...] += part

    @pl.when(i == nb - 1)
    def _():
        h0 = acc_ref[...]
        h0_ref[...] = h0
        sh0_ref[...] = jnp.dot(h0, wa1_ref[...],
                               preferred_element_type=jnp.float32,
                               precision=_HI)


def _loop_body(wn_ref, sn_ref, bi_ref, sh0_ref, h0_ref,
               gk_ref, gr_ref, gb_ref, wa1_ref, hn_ref,
               acc_ref, h_scr, sh_scr):
    d = pl.program_id(0)
    i = pl.program_id(1)
    nb = pl.num_programs(1)

    @pl.when((d == 0) & (i == 0))
    def _():
        h_scr[...] = h0_ref[...]
        sh_scr[...] = sh0_ref[...]

    bi = bi_ref[0]
    mask = jax.lax.broadcasted_iota(jnp.int32, (BATCH, BLK), 0) == bi
    oh = mask.astype(jnp.bfloat16)
    sh = sh_scr[...]
    shh = sh.astype(jnp.bfloat16)
    shl = (sh - shh.astype(jnp.float32)).astype(jnp.bfloat16)
    shg = _f32(shh, oh) + _f32(shl, oh)
    av = shg[0:1] + sn_ref[0]
    av = jnp.where(av > 0, av, 0.2 * av)
    e = jnp.exp(av)
    ow = jnp.where(mask, jnp.broadcast_to(e, (BATCH, BLK)),
                   0.0).astype(jnp.bfloat16)
    part = _f32(ow, wn_ref[...])

    @pl.when(i == 0)
    def _():
        acc_ref[...] = part

    @pl.when(i > 0)
    def _():
        acc_ref[...] += part

    @pl.when(i == nb - 1)
    def _():
        acc = acc_ref[...]
        denom = jnp.maximum(acc[:, UNITS:UNITS + 1], 1e-30)
        cont = acc[:, :UNITS] / denom
        cont = jnp.where(cont > 0, cont, jnp.exp(cont) - 1.0)
        h = h_scr[...]
        mx = _f32(cont, gk_ref[...]) + gb_ref[0:1]
        mi = _f32(h, gr_ref[...]) + gb_ref[1:2]
        xz, xr, xg = mx[:, :UNITS], mx[:, UNITS:2 * UNITS], mx[:, 2 * UNITS:]
        rz, rr, rg = mi[:, :UNITS], mi[:, UNITS:2 * UNITS], mi[:, 2 * UNITS:]
        z = jax.nn.sigmoid(xz + rz)
        r = jax.nn.sigmoid(xr + rr)
        hh = jnp.tanh(xg + r * rg)
        hn = z * h + (1.0 - z) * hh
        h_scr[...] = hn
        shc = jnp.dot(hn, wa1_ref[...], preferred_element_type=jnp.float32,
                      precision=_HI)
        sh_scr[...] = jnp.broadcast_to(shc[:, 0].reshape(1, BATCH), (8, BATCH))

        @pl.when(d == DEPTH - 1)
        def _():
            hn_ref[...] = hn


def kernel(ref, node, batch_index, W_lin, b_lin, W_alpha, b_alpha,
           gru_kernel, gru_rec, gru_bias):
    N = node.shape[0]
    NB = -(-N // BLK)
    NPAD = NB * BLK
    nodep = jnp.pad(node, ((0, NPAD - N), (0, 0)))
    bip = jnp.pad(batch_index.astype(jnp.int32), (0, NPAD - N),
                  constant_values=BATCH)
    bi3 = bip.reshape(NB, 1, BLK)

    W_aug = jnp.zeros((F, AW), jnp.float32).at[:, :UNITS].set(W_lin)
    b_aug = (jnp.zeros((AW,), jnp.float32)
             .at[:UNITS].set(b_lin)
             .at[UNITS].set(1.0)).reshape(1, AW)
    wa2 = jnp.zeros((F, 128), jnp.float32).at[:, 0].set(W_alpha[UNITS:, 0])
    w2h = wa2.astype(jnp.bfloat16)
    w2l = (wa2 - w2h.astype(jnp.float32)).astype(jnp.bfloat16)
    ba_row = jnp.broadcast_to(b_alpha.reshape(1, 1), (1, 128))
    wa1 = jnp.zeros((UNITS, 128), jnp.float32).at[:, 0].set(W_alpha[:UNITS, 0])
    gbp = jnp.zeros((8, 3 * UNITS), jnp.float32).at[:2].set(gru_bias)

    def full(shape):
        nzero = len(shape)
        return pl.BlockSpec(shape, lambda *args, _n=nzero: (0,) * _n)

    wn_aug, sn_col, h0, sh_col = pl.pallas_call(
        _prep_body,
        grid=(NB,),
        in_specs=[
            pl.BlockSpec((BLK, F), lambda i: (i, 0)),
            full((F, AW)),
            full((1, AW)),
            full((F, 128)),
            full((F, 128)),
            full((1, 128)),
            full((UNITS, 128)),
            pl.BlockSpec((1, 1, BLK), lambda i: (i, 0, 0)),
        ],
        out_specs=[
            pl.BlockSpec((BLK, AW), lambda i: (i, 0)),
            pl.BlockSpec((BLK, 128), lambda i: (i, 0)),
            full((BATCH, F)),
            full((BATCH, 128)),
        ],
        out_shape=[
            jax.ShapeDtypeStruct((NPAD, AW), jnp.bfloat16),
            jax.ShapeDtypeStruct((NPAD, 128), jnp.float32),
            jax.ShapeDtypeStruct((BATCH, F), jnp.float32),
            jax.ShapeDtypeStruct((BATCH, 128), jnp.float32),
        ],
        scratch_shapes=[pltpu.VMEM((BATCH, F), jnp.float32)],
    )(nodep, W_aug, b_aug, w2h, w2l, ba_row, wa1, bi3)

    sn_row = sn_col[:, 0].reshape(NB, 1, BLK)
    sh08 = jnp.broadcast_to(sh_col[:, 0].reshape(1, BATCH), (8, BATCH))

    h = pl.pallas_call(
        _loop_body,
        grid=(DEPTH, NB),
        in_specs=[
            pl.BlockSpec((BLK, AW), lambda d, i: (i, 0)),
            pl.BlockSpec((1, 1, BLK), lambda d, i: (i, 0, 0)),
            pl.BlockSpec((1, 1, BLK), lambda d, i: (i, 0, 0)),
            full((8, BATCH)),
            full((BATCH, UNITS)),
            full((UNITS, 3 * UNITS)),
            full((UNITS, 3 * UNITS)),
            full((8, 3 * UNITS)),
            full((UNITS, 128)),
        ],
        out_specs=full((BATCH, UNITS)),
        out_shape=jax.ShapeDtypeStruct((BATCH, UNITS), jnp.float32),
        scratch_shapes=[
            pltpu.VMEM((BATCH, AW), jnp.float32),
            pltpu.VMEM((BATCH, UNITS), jnp.float32),
            pltpu.VMEM((8, BATCH), jnp.float32),
        ],
    )(wn_aug, sn_row, bi3, sh08, h0, gru_kernel, gru_rec, gbp, wa1)
    return h
